# baseline (speedup 1.0000x reference)
import functools

import jax
import jax.numpy as jnp
from jax import lax
from jax.experimental import pallas as pl
from jax.experimental.pallas import tpu as pltpu
from jax.experimental.pallas import tpu_sc as plsc

_N, _M, _K = 4096, 20, 1000
_NC, _NS, _L = 2, 16, 16
_NW = _NC * _NS
_SLICES_PER_W = _N // _NW
_C = 2
_CHUNKS = _SLICES_PER_W // _C
_ROWS = _C * _M
_IDX_PER_W = _SLICES_PER_W * _M
_IDX_PAD = _IDX_PER_W + _L
_NVEC = (_ROWS + _L - 1) // _L


def _scatter_chunk(buf, idx_v, base, value):
    vals = jnp.full((_L,), value, jnp.float32)
    for p in range(_NVEC):
        t = lax.iota(jnp.int32, _L) + p * _L
        valid = t < _ROWS
        v = idx_v[pl.ds(base + p * _L, _L)]
        a = (t >= _M).astype(jnp.int32) + (t >= 2 * _M).astype(jnp.int32)
        a = jnp.minimum(a, _C - 1)
        j = t - a * _M
        j = jnp.minimum(j, _M - 1)
        plsc.store_scatter(buf, [a, j, v], vals, mask=valid)


def _zero_buf(buf):
    zeros = jnp.zeros((_L,), jnp.float32)
    offs = tuple(min(o * _L, _K - _L) for o in range((_K + _L - 1) // _L))

    def row(r, carry):
        a = r // _M
        j = r - a * _M
        for o in offs:
            buf[a, j, pl.ds(o, _L)] = zeros
        return carry

    lax.fori_loop(0, _ROWS, row, 0)


def _sc_body(x_ref, out_ref, idx_v, buf_a, buf_b, sem_a, sem_b):
    wid = lax.axis_index("s") * _NC + lax.axis_index("c")
    slice0 = wid * _SLICES_PER_W

    pltpu.sync_copy(x_ref.at[pl.ds(wid * _IDX_PER_W, _IDX_PER_W)],
                    idx_v.at[pl.ds(0, _IDX_PER_W)])
    _zero_buf(buf_a)
    _zero_buf(buf_b)

    def start(c, buf, sem):
        _scatter_chunk(buf, idx_v, c * _ROWS, 1.0)
        dst = out_ref.at[pl.ds(slice0 + c * _C, _C)]
        pltpu.async_copy(buf, dst, sem)

    def drain(c, buf, sem):
        dst = out_ref.at[pl.ds(slice0 + c * _C, _C)]
        pltpu.make_async_copy(buf, dst, sem).wait()
        _scatter_chunk(buf, idx_v, c * _ROWS, 0.0)

    start(0, buf_a, sem_a)
    start(1, buf_b, sem_b)

    def step(c2, carry):
        for b, (buf, sem) in enumerate(((buf_a, sem_a), (buf_b, sem_b))):
            c = 2 * c2 + b
            drain(c - 2, buf, sem)
            start(c, buf, sem)
        return carry

    lax.fori_loop(1, _CHUNKS // 2, step, 0)
    drain(_CHUNKS - 2, buf_a, sem_a)
    drain(_CHUNKS - 1, buf_b, sem_b)


def kernel(x, table):
    del table
    xf = jnp.reshape(x, (_N * _M,)).astype(jnp.int32)
    mesh = plsc.VectorSubcoreMesh(core_axis_name="c", subcore_axis_name="s")
    f = functools.partial(
        pl.kernel,
        out_type=jax.ShapeDtypeStruct((_N, _M, _K), jnp.float32),
        mesh=mesh,
        compiler_params=pltpu.CompilerParams(
            use_tc_tiling_on_sc=False, needs_layout_passes=False),
        scratch_types=[
            pltpu.VMEM((_IDX_PAD,), jnp.int32),
            pltpu.VMEM((_C, _M, _K), jnp.float32),
            pltpu.VMEM((_C, _M, _K), jnp.float32),
            pltpu.SemaphoreType.DMA,
            pltpu.SemaphoreType.DMA,
        ],
    )(_sc_body)
    return f(xf)


# speedup vs baseline: 1.4771x; 1.4771x over previous
import functools

import jax
import jax.numpy as jnp
from jax import lax
from jax.experimental import pallas as pl
from jax.experimental.pallas import tpu as pltpu
from jax.experimental.pallas import tpu_sc as plsc

_N, _M, _K = 4096, 20, 1000
_NC, _NS, _L = 2, 16, 16
_NW = _NC * _NS
_SLICES_PER_W = _N // _NW
_C = 2
_CHUNKS = _SLICES_PER_W // _C
_ROWS = _C * _M
_IDX_PER_W = _SLICES_PER_W * _M
_IDX_PAD = _IDX_PER_W + _L
_NVEC = (_ROWS + _L - 1) // _L


def _scatter_chunk(buf, idx_v, base, value):
    vals = jnp.full((_L,), value, jnp.float32)
    for p in range(_NVEC):
        t = lax.iota(jnp.int32, _L) + p * _L
        valid = t < _ROWS
        v = idx_v[pl.ds(base + p * _L, _L)]
        a = (t >= _M).astype(jnp.int32) + (t >= 2 * _M).astype(jnp.int32)
        a = jnp.minimum(a, _C - 1)
        j = t - a * _M
        j = jnp.minimum(j, _M - 1)
        plsc.store_scatter(buf, [a, j, v], vals, mask=valid)


def _zero_buf(buf):
    zeros = jnp.zeros((_L,), jnp.float32)
    offs = tuple(min(o * _L, _K - _L) for o in range((_K + _L - 1) // _L))

    def row(r, carry):
        a = r // _M
        j = r - a * _M
        for o in offs:
            buf[a, j, pl.ds(o, _L)] = zeros
        return carry

    lax.fori_loop(0, _ROWS, row, 0)


def _sc_body(x_ref, out_ref, idx_v, buf_a, buf_b, sem_a, sem_b):
    wid = lax.axis_index("s") * _NC + lax.axis_index("c")
    slice0 = wid * _SLICES_PER_W

    pltpu.sync_copy(x_ref.at[pl.ds(wid * _IDX_PER_W, _IDX_PER_W)],
                    idx_v.at[pl.ds(0, _IDX_PER_W)])
    _zero_buf(buf_a)
    _zero_buf(buf_b)

    def start(c, buf, sem):
        _scatter_chunk(buf, idx_v, c * _ROWS, 1.0)
        dst = out_ref.at[pl.ds(slice0 + c * _C, _C)]
        pltpu.async_copy(buf, dst, sem)

    def drain(c, buf, sem):
        dst = out_ref.at[pl.ds(slice0 + c * _C, _C)]
        pltpu.make_async_copy(buf, dst, sem).wait()
        _scatter_chunk(buf, idx_v, c * _ROWS, 0.0)

    start(0, buf_a, sem_a)
    start(1, buf_b, sem_b)

    def step(c2, carry):
        for b, (buf, sem) in enumerate(((buf_a, sem_a), (buf_b, sem_b))):
            c = 2 * c2 + b
            drain(c - 2, buf, sem)
            start(c, buf, sem)
        return carry

    lax.fori_loop(1, _CHUNKS // 2, step, 0)
    drain(_CHUNKS - 2, buf_a, sem_a)
    drain(_CHUNKS - 1, buf_b, sem_b)


def kernel(x, table):
    del table
    xf = jnp.reshape(x, (_N * _M,)).astype(jnp.int32)
    mesh = plsc.VectorSubcoreMesh(core_axis_name="c", subcore_axis_name="s")
    f = functools.partial(
        pl.kernel,
        out_type=jax.ShapeDtypeStruct((_N, _M, _K), jnp.float32),
        mesh=mesh,
        compiler_params=pltpu.CompilerParams(
            needs_layout_passes=False),
        scratch_types=[
            pltpu.VMEM((_IDX_PAD,), jnp.int32),
            pltpu.VMEM((_C, _M, _K), jnp.float32),
            pltpu.VMEM((_C, _M, _K), jnp.float32),
            pltpu.SemaphoreType.DMA,
            pltpu.SemaphoreType.DMA,
        ],
    )(_sc_body)
    return f(xf)
